# Initial kernel scaffold; baseline (speedup 1.0000x reference)
import jax
import jax.numpy as jnp
from jax.experimental import pallas as pl


def kernel(x, W):
    raise NotImplementedError("write your pallas kernel here")


# speedup vs baseline: 3.3753x; 3.3753x over previous
import functools

import jax
import jax.numpy as jnp
from jax import lax
from jax.experimental import pallas as pl
from jax.experimental.pallas import tpu as pltpu
from jax.experimental.pallas import tpu_sc as plsc

_NC = 2
_NS = 16
_NW = _NC * _NS
_L = 16
_CH = 256


def _sc_expand(table, idx):
    n_total = idx.shape[0]
    nrows, depth = table.shape
    pad = 128
    table_p = jnp.pad(table, ((0, 0), (0, pad - depth)))
    b_per_w = n_total // _NW
    n_chunks = b_per_w // _CH
    n_sub = b_per_w // pad
    idx3 = idx.reshape(_NW, n_sub, pad)
    sub_per_ch = _CH // pad

    offs = [o * _L for o in range(depth // _L)]
    if depth % _L:
        offs.append(depth - _L)

    mesh = plsc.VectorSubcoreMesh(core_axis_name="c", subcore_axis_name="s")

    @functools.partial(
        pl.kernel,
        mesh=mesh,
        out_type=jax.ShapeDtypeStruct((n_total, depth), jnp.float32),
        scratch_types=[
            pltpu.VMEM((nrows, pad), jnp.float32),
            pltpu.VMEM((n_sub, pad), jnp.int32),
            pltpu.VMEM((_CH, depth), jnp.float32),
            pltpu.VMEM((_CH, depth), jnp.float32),
            pltpu.SemaphoreType.DMA,
            pltpu.SemaphoreType.DMA,
        ],
    )
    def k(table_hbm, idx_hbm, out_hbm, tab_v, idx_v, buf0, buf1, sem0, sem1):
        wid = lax.axis_index("s") * _NC + lax.axis_index("c")
        base = wid * b_per_w
        pltpu.sync_copy(table_hbm, tab_v)
        pltpu.sync_copy(idx_hbm.at[wid], idx_v)

        bufs = ((buf0, sem0), (buf1, sem1))

        def fill(buf, c):
            for h in range(sub_per_ch):
                def grp(g, carry, h=h):
                    vvec = idx_v[c * sub_per_ch + h, pl.ds(g * _L, _L)]
                    for l in range(_L):
                        v = vvec[l]
                        dr = h * pad + g * _L + l
                        for o in offs:
                            buf[dr, pl.ds(o, _L)] = tab_v[v, pl.ds(o, _L)]
                    return carry
                lax.fori_loop(0, pad // _L, grp, 0)

        def chunk_pair(p, carry):
            for b, (buf, sem) in enumerate(bufs):
                c = p * 2 + b
                dst = out_hbm.at[pl.ds(base + c * _CH, _CH)]

                @pl.when(c >= 2)
                def _():
                    pltpu.make_async_copy(buf, dst, sem).wait()

                fill(buf, c)
                pltpu.async_copy(buf, dst, sem)
            return carry

        lax.fori_loop(0, n_chunks // 2, chunk_pair, 0)

        for b, (buf, sem) in enumerate(bufs):
            c = n_chunks - 2 + b
            dst = out_hbm.at[pl.ds(base + c * _CH, _CH)]
            pltpu.make_async_copy(buf, dst, sem).wait()

    return k(table_p, idx3)


def kernel(x, W):
    n0, n1 = x.shape
    out = _sc_expand(W, x.reshape(-1))
    return out.reshape(n0, n1, W.shape[1])
